# baseline (speedup 1.0000x reference)
import functools

import jax
import jax.numpy as jnp
from jax import lax
from jax.experimental import pallas as pl
from jax.experimental.pallas import tpu as pltpu
from jax.experimental.pallas import tpu_sc as plsc

CHUNK = 128
NBUF = 4
LA = 2


@functools.partial(jax.jit, static_argnames=("v",))
def _sc_relayout(wt, *, v):
    d = wt.shape[0]
    assert v % 128 == 0
    n_blocks = v // 128
    per_w = (n_blocks + 31) // 32
    mesh = plsc.VectorSubcoreMesh(core_axis_name="c", subcore_axis_name="s")

    @functools.partial(
        pl.kernel,
        mesh=mesh,
        compiler_params=pltpu.CompilerParams(
            use_tc_tiling_on_sc=True, needs_layout_passes=False),
        out_type=jax.ShapeDtypeStruct((v, 128), jnp.float32),
        scratch_types=(
            [pltpu.VMEM((2, d, 128), jnp.float32),
             pltpu.VMEM((2, 128, 128), jnp.float32)]
            + [pltpu.SemaphoreType.DMA] * 4
        ),
    )
    def k(wt_hbm, out_hbm, blk_v, row_v, *sems):
        lsem = sems[:2]
        ssem = sems[2:]
        wid = lax.axis_index("s") * 2 + lax.axis_index("c")
        c0 = wid * per_w
        iota = lax.iota(jnp.int32, 16)

        def load_block(c, buf):
            @pl.when(c < n_blocks)
            def _():
                pltpu.async_copy(
                    wt_hbm.at[:, pl.ds(c * 128, 128)], blk_v.at[buf], lsem[buf]
                )

        def wait_load(c, buf):
            @pl.when(c < n_blocks)
            def _():
                pltpu.make_async_copy(
                    wt_hbm.at[:, pl.ds(0, 128)], blk_v.at[buf], lsem[buf]
                ).wait()

        def store_block(c, buf):
            @pl.when(c < n_blocks)
            def _():
                pltpu.async_copy(
                    row_v.at[buf], out_hbm.at[pl.ds(c * 128, 128)], ssem[buf]
                )

        def wait_store(c, buf):
            @pl.when(jnp.logical_and(c >= c0, c < n_blocks))
            def _():
                pltpu.make_async_copy(
                    row_v.at[buf], out_hbm.at[pl.ds(0, 128)], ssem[buf]
                ).wait()

        load_block(c0, 0)
        n_iter = per_w + (per_w % 2)

        def body(g, _):
            for bb in range(2):
                i = g * 2 + bb
                c = c0 + i
                load_block(c + 1, 1 - bb)
                wait_load(c, bb)
                wait_store(c - 2, bb)

                @pl.when(c < n_blocks)
                def _():
                    def trans(t, _):
                        for j in range(d // 16):
                            vals = plsc.load_gather(
                                blk_v.at[bb],
                                [j * 16 + iota, jnp.full((16,), t, jnp.int32)],
                            )
                            row_v[bb, t, pl.ds(j * 16, 16)] = vals
                        return 0

                    lax.fori_loop(0, 128, trans, 0, unroll=4)
                    store_block(c, bb)

            return 0

        lax.fori_loop(0, n_iter // 2, body, 0)
        wait_load(c0 + n_iter, 0)
        wait_store(c0 + n_iter - 2, 0)
        wait_store(c0 + n_iter - 1, 1)

    return k(wt)


@functools.partial(jax.jit, static_argnames=("n_chunks", "d"))
def _sc_gather(table, idx3, *, n_chunks, d):
    nw = idx3.shape[0]
    b = nw * n_chunks * CHUNK
    mesh = plsc.VectorSubcoreMesh(core_axis_name="c", subcore_axis_name="s")

    @functools.partial(
        pl.kernel,
        mesh=mesh,
        compiler_params=pltpu.CompilerParams(use_tc_tiling_on_sc=True),
        out_type=jax.ShapeDtypeStruct((b, 128), jnp.float32),
        scratch_types=(
            [pltpu.VMEM((n_chunks, CHUNK), jnp.int32),
             pltpu.VMEM((NBUF, CHUNK, 128), jnp.float32)]
            + [pltpu.SemaphoreType.DMA] * (2 * NBUF)
        ),
    )
    def k(table_hbm, idx_hbm, out_hbm, idx_v, rows_v, *sems):
        gsem = sems[:NBUF]
        ssem = sems[NBUF:]
        wid = lax.axis_index("s") * 2 + lax.axis_index("c")
        base = wid * (n_chunks * CHUNK)
        pltpu.sync_copy(idx_hbm.at[wid], idx_v)

        for jj in range(LA):
            pltpu.async_copy(table_hbm.at[idx_v.at[jj]], rows_v.at[jj], gsem[jj])

        def outer(g, _):
            j0 = g * NBUF
            for bb in range(NBUF):
                j = j0 + bb
                jl = j + LA
                bl = (bb + LA) % NBUF

                @pl.when(jl < n_chunks)
                def _():
                    @pl.when(jl >= NBUF)
                    def _():
                        pltpu.make_async_copy(
                            rows_v.at[bl],
                            out_hbm.at[pl.ds(base, CHUNK)],
                            ssem[bl],
                        ).wait()

                    pltpu.async_copy(
                        table_hbm.at[idx_v.at[jl]], rows_v.at[bl], gsem[bl]
                    )

                pltpu.make_async_copy(
                    table_hbm.at[idx_v.at[j]], rows_v.at[bb], gsem[bb]
                ).wait()
                pltpu.async_copy(
                    rows_v.at[bb],
                    out_hbm.at[pl.ds(base + j * CHUNK, CHUNK)],
                    ssem[bb],
                )
            return 0

        lax.fori_loop(0, n_chunks // NBUF, outer, 0)

        for bb in range(NBUF):
            pltpu.make_async_copy(
                rows_v.at[bb], out_hbm.at[pl.ds(base, CHUNK)], ssem[bb]
            ).wait()

    return k(table, idx3)


def kernel(token_ids, weight):
    bt, s = token_ids.shape
    v, d = weight.shape
    flat = token_ids.reshape(-1).astype(jnp.int32)
    b = flat.shape[0]
    nw = 32
    per_w = b // nw
    n_chunks = per_w // CHUNK
    idx3 = flat.reshape(nw, n_chunks, CHUNK)
    v_pad = (v + 127) // 128 * 128
    wp = jnp.pad(weight, ((0, v_pad - v), (0, 0)))
    table = _sc_relayout(wp.T, v=v_pad)
    out = _sc_gather(table, idx3, n_chunks=n_chunks, d=d)
    return out[:, :d].reshape(bt, s, d)


# speedup vs baseline: 1.1533x; 1.1533x over previous
import functools

import jax
import jax.numpy as jnp
from jax import lax
from jax.experimental import pallas as pl
from jax.experimental.pallas import tpu as pltpu
from jax.experimental.pallas import tpu_sc as plsc

CHUNK = 128
NBUF = 4
LA = 2


@functools.partial(jax.jit, static_argnames=("v",))
def _sc_relayout(wt, *, v):
    d = wt.shape[0]
    assert v % 128 == 0
    n_blocks = v // 128
    per_w = (n_blocks + 31) // 32
    mesh = plsc.VectorSubcoreMesh(core_axis_name="c", subcore_axis_name="s")

    @functools.partial(
        pl.kernel,
        mesh=mesh,
        compiler_params=pltpu.CompilerParams(
            use_tc_tiling_on_sc=True, needs_layout_passes=False),
        out_type=jax.ShapeDtypeStruct((v, 128), jnp.float32),
        scratch_types=(
            [pltpu.VMEM((2, d, 128), jnp.float32),
             pltpu.VMEM((2, 128, 128), jnp.float32)]
            + [pltpu.SemaphoreType.DMA] * 4
        ),
    )
    def k(wt_hbm, out_hbm, blk_v, row_v, *sems):
        lsem = sems[:2]
        ssem = sems[2:]
        wid = lax.axis_index("s") * 2 + lax.axis_index("c")
        c0 = wid * per_w
        iota = lax.iota(jnp.int32, 16)

        def load_block(c, buf):
            @pl.when(c < n_blocks)
            def _():
                pltpu.async_copy(
                    wt_hbm.at[:, pl.ds(c * 128, 128)], blk_v.at[buf], lsem[buf]
                )

        def wait_load(c, buf):
            @pl.when(c < n_blocks)
            def _():
                pltpu.make_async_copy(
                    wt_hbm.at[:, pl.ds(0, 128)], blk_v.at[buf], lsem[buf]
                ).wait()

        def store_block(c, buf):
            @pl.when(c < n_blocks)
            def _():
                pltpu.async_copy(
                    row_v.at[buf], out_hbm.at[pl.ds(c * 128, 128)], ssem[buf]
                )

        def wait_store(c, buf):
            @pl.when(jnp.logical_and(c >= c0, c < n_blocks))
            def _():
                pltpu.make_async_copy(
                    row_v.at[buf], out_hbm.at[pl.ds(0, 128)], ssem[buf]
                ).wait()

        load_block(c0, 0)
        n_iter = per_w + (per_w % 2)

        def body(g, _):
            for bb in range(2):
                i = g * 2 + bb
                c = c0 + i
                load_block(c + 1, 1 - bb)
                wait_load(c, bb)
                wait_store(c - 2, bb)

                @pl.when(c < n_blocks)
                def _():
                    def trans(f, _):
                        colv = jnp.full((16,), f, jnp.int32)
                        for m in range(8):
                            vals = blk_v[bb, f, pl.ds(m * 16, 16)]
                            plsc.store_scatter(
                                row_v.at[bb], [m * 16 + iota, colv], vals
                            )
                        return 0

                    lax.fori_loop(0, d, trans, 0, unroll=2)
                    store_block(c, bb)

            return 0

        lax.fori_loop(0, n_iter // 2, body, 0)
        wait_load(c0 + n_iter, 0)
        wait_store(c0 + n_iter - 2, 0)
        wait_store(c0 + n_iter - 1, 1)

    return k(wt)


@functools.partial(jax.jit, static_argnames=("n_chunks", "d"))
def _sc_gather(table, idx3, *, n_chunks, d):
    nw = idx3.shape[0]
    b = nw * n_chunks * CHUNK
    mesh = plsc.VectorSubcoreMesh(core_axis_name="c", subcore_axis_name="s")

    @functools.partial(
        pl.kernel,
        mesh=mesh,
        compiler_params=pltpu.CompilerParams(use_tc_tiling_on_sc=True),
        out_type=jax.ShapeDtypeStruct((b, 128), jnp.float32),
        scratch_types=(
            [pltpu.VMEM((n_chunks, CHUNK), jnp.int32),
             pltpu.VMEM((NBUF, CHUNK, 128), jnp.float32)]
            + [pltpu.SemaphoreType.DMA] * (2 * NBUF)
        ),
    )
    def k(table_hbm, idx_hbm, out_hbm, idx_v, rows_v, *sems):
        gsem = sems[:NBUF]
        ssem = sems[NBUF:]
        wid = lax.axis_index("s") * 2 + lax.axis_index("c")
        base = wid * (n_chunks * CHUNK)
        pltpu.sync_copy(idx_hbm.at[wid], idx_v)

        for jj in range(LA):
            pltpu.async_copy(table_hbm.at[idx_v.at[jj]], rows_v.at[jj], gsem[jj])

        def outer(g, _):
            j0 = g * NBUF
            for bb in range(NBUF):
                j = j0 + bb
                jl = j + LA
                bl = (bb + LA) % NBUF

                @pl.when(jl < n_chunks)
                def _():
                    @pl.when(jl >= NBUF)
                    def _():
                        pltpu.make_async_copy(
                            rows_v.at[bl],
                            out_hbm.at[pl.ds(base, CHUNK)],
                            ssem[bl],
                        ).wait()

                    pltpu.async_copy(
                        table_hbm.at[idx_v.at[jl]], rows_v.at[bl], gsem[bl]
                    )

                pltpu.make_async_copy(
                    table_hbm.at[idx_v.at[j]], rows_v.at[bb], gsem[bb]
                ).wait()
                pltpu.async_copy(
                    rows_v.at[bb],
                    out_hbm.at[pl.ds(base + j * CHUNK, CHUNK)],
                    ssem[bb],
                )
            return 0

        lax.fori_loop(0, n_chunks // NBUF, outer, 0)

        for bb in range(NBUF):
            pltpu.make_async_copy(
                rows_v.at[bb], out_hbm.at[pl.ds(base, CHUNK)], ssem[bb]
            ).wait()

    return k(table, idx3)


def kernel(token_ids, weight):
    bt, s = token_ids.shape
    v, d = weight.shape
    flat = token_ids.reshape(-1).astype(jnp.int32)
    b = flat.shape[0]
    nw = 32
    per_w = b // nw
    n_chunks = per_w // CHUNK
    idx3 = flat.reshape(nw, n_chunks, CHUNK)
    v_pad = (v + 127) // 128 * 128
    wp = jnp.pad(weight, ((0, v_pad - v), (0, 0)))
    table = _sc_relayout(wp.T, v=v_pad)
    out = _sc_gather(table, idx3, n_chunks=n_chunks, d=d)
    return out[:, :d].reshape(bt, s, d)


# speedup vs baseline: 2.4658x; 2.1380x over previous
import functools

import jax
import jax.numpy as jnp
from jax import lax
from jax.experimental import pallas as pl
from jax.experimental.pallas import tpu as pltpu
from jax.experimental.pallas import tpu_sc as plsc

CHUNK = 128
NBUF = 4
LA = 2


@functools.partial(jax.jit, static_argnames=("v",))
def _sc_relayout(wt, *, v):
    d = wt.shape[0]
    assert v % 128 == 0
    n_blocks = v // 128
    per_w = (n_blocks + 31) // 32
    mesh = plsc.VectorSubcoreMesh(core_axis_name="c", subcore_axis_name="s")

    @functools.partial(
        pl.kernel,
        mesh=mesh,
        compiler_params=pltpu.CompilerParams(
            use_tc_tiling_on_sc=True, needs_layout_passes=False),
        out_type=jax.ShapeDtypeStruct((v, 128), jnp.float32),
        scratch_types=(
            [pltpu.VMEM((2, d, 128), jnp.float32),
             pltpu.VMEM((2, 128, 128), jnp.float32)]
            + [pltpu.SemaphoreType.DMA] * 4
        ),
    )
    def k(wt_hbm, out_hbm, blk_v, row_v, *sems):
        lsem = sems[:2]
        ssem = sems[2:]
        wid = lax.axis_index("s") * 2 + lax.axis_index("c")
        c0 = wid * per_w
        iota = lax.iota(jnp.int32, 16)

        def load_block(c, buf):
            @pl.when(c < n_blocks)
            def _():
                pltpu.async_copy(
                    wt_hbm.at[:, pl.ds(c * 128, 128)], blk_v.at[buf], lsem[buf]
                )

        def wait_load(c, buf):
            @pl.when(c < n_blocks)
            def _():
                pltpu.make_async_copy(
                    wt_hbm.at[:, pl.ds(0, 128)], blk_v.at[buf], lsem[buf]
                ).wait()

        def store_block(c, buf):
            @pl.when(c < n_blocks)
            def _():
                pltpu.async_copy(
                    row_v.at[buf], out_hbm.at[pl.ds(c * 128, 128)], ssem[buf]
                )

        def wait_store(c, buf):
            @pl.when(jnp.logical_and(c >= c0, c < n_blocks))
            def _():
                pltpu.make_async_copy(
                    row_v.at[buf], out_hbm.at[pl.ds(0, 128)], ssem[buf]
                ).wait()

        load_block(c0, 0)
        n_iter = per_w + (per_w % 2)

        def body(g, _):
            for bb in range(2):
                i = g * 2 + bb
                c = c0 + i
                load_block(c + 1, 1 - bb)
                wait_load(c, bb)
                wait_store(c - 2, bb)

                @pl.when(c < n_blocks)
                def _():
                    def trans(f, _):
                        colv = jnp.full((16,), f, jnp.int32)
                        for m in range(8):
                            vals = blk_v[bb, f, pl.ds(m * 16, 16)]
                            plsc.store_scatter(
                                row_v.at[bb], [m * 16 + iota, colv], vals
                            )
                        return 0

                    lax.fori_loop(0, d, trans, 0, unroll=2)
                    store_block(c, bb)

            return 0

        lax.fori_loop(0, n_iter // 2, body, 0)
        wait_load(c0 + n_iter, 0)
        wait_store(c0 + n_iter - 2, 0)
        wait_store(c0 + n_iter - 1, 1)

    return k(wt)


@functools.partial(jax.jit, static_argnames=("n_chunks", "d"))
def _sc_gather(table, idx3, *, n_chunks, d):
    nw = idx3.shape[0]
    b = nw * n_chunks * CHUNK
    mesh = plsc.VectorSubcoreMesh(core_axis_name="c", subcore_axis_name="s")

    @functools.partial(
        pl.kernel,
        mesh=mesh,
        compiler_params=pltpu.CompilerParams(use_tc_tiling_on_sc=False),
        out_type=jax.ShapeDtypeStruct((b, 128), jnp.float32),
        scratch_types=(
            [pltpu.VMEM((n_chunks, CHUNK), jnp.int32),
             pltpu.VMEM((NBUF, CHUNK, d), jnp.float32)]
            + [pltpu.SemaphoreType.DMA] * (2 * NBUF)
        ),
    )
    def k(table_hbm, idx_hbm, out_hbm, idx_v, rows_v, *sems):
        gsem = sems[:NBUF]
        ssem = sems[NBUF:]
        wid = lax.axis_index("s") * 2 + lax.axis_index("c")
        base = wid * (n_chunks * CHUNK)
        pltpu.sync_copy(idx_hbm.at[wid], idx_v)

        for jj in range(LA):
            pltpu.async_copy(table_hbm.at[idx_v.at[jj]], rows_v.at[jj], gsem[jj])

        def outer(g, _):
            j0 = g * NBUF
            for bb in range(NBUF):
                j = j0 + bb
                jl = j + LA
                bl = (bb + LA) % NBUF

                @pl.when(jl < n_chunks)
                def _():
                    @pl.when(jl >= NBUF)
                    def _():
                        pltpu.make_async_copy(
                            rows_v.at[bl],
                            out_hbm.at[pl.ds(base, CHUNK), pl.ds(0, d)],
                            ssem[bl],
                        ).wait()

                    pltpu.async_copy(
                        table_hbm.at[idx_v.at[jl]], rows_v.at[bl], gsem[bl]
                    )

                pltpu.make_async_copy(
                    table_hbm.at[idx_v.at[j]], rows_v.at[bb], gsem[bb]
                ).wait()
                pltpu.async_copy(
                    rows_v.at[bb],
                    out_hbm.at[pl.ds(base + j * CHUNK, CHUNK), pl.ds(0, d)],
                    ssem[bb],
                )
            return 0

        lax.fori_loop(0, n_chunks // NBUF, outer, 0)

        for bb in range(NBUF):
            pltpu.make_async_copy(
                rows_v.at[bb], out_hbm.at[pl.ds(base, CHUNK), pl.ds(0, d)], ssem[bb]
            ).wait()

    return k(table, idx3)


def kernel(token_ids, weight):
    bt, s = token_ids.shape
    v, d = weight.shape
    flat = token_ids.reshape(-1).astype(jnp.int32)
    b = flat.shape[0]
    nw = 32
    per_w = b // nw
    n_chunks = per_w // CHUNK
    idx3 = (flat * 2).reshape(nw, n_chunks, CHUNK)
    table = jnp.pad(weight, ((0, 0), (0, 128 - d))).reshape(2 * v, d)
    out = _sc_gather(table, idx3, n_chunks=n_chunks, d=d)
    return out[:, :d].reshape(bt, s, d)


# speedup vs baseline: 2.4697x; 1.0016x over previous
import functools

import jax
import jax.numpy as jnp
from jax import lax
from jax.experimental import pallas as pl
from jax.experimental.pallas import tpu as pltpu
from jax.experimental.pallas import tpu_sc as plsc

CHUNK = 128
NBUF = 8
LA = 4


@functools.partial(jax.jit, static_argnames=("n_chunks", "d"))
def _sc_gather(table, idx3, *, n_chunks, d):
    nw = idx3.shape[0]
    b = nw * n_chunks * CHUNK
    mesh = plsc.VectorSubcoreMesh(core_axis_name="c", subcore_axis_name="s")

    @functools.partial(
        pl.kernel,
        mesh=mesh,
        compiler_params=pltpu.CompilerParams(use_tc_tiling_on_sc=False),
        out_type=jax.ShapeDtypeStruct((b, 128), jnp.float32),
        scratch_types=(
            [pltpu.VMEM((n_chunks, CHUNK), jnp.int32),
             pltpu.VMEM((NBUF, CHUNK, d), jnp.float32)]
            + [pltpu.SemaphoreType.DMA] * (2 * NBUF)
        ),
    )
    def k(table_hbm, idx_hbm, out_hbm, idx_v, rows_v, *sems):
        gsem = sems[:NBUF]
        ssem = sems[NBUF:]
        wid = lax.axis_index("s") * 2 + lax.axis_index("c")
        base = wid * (n_chunks * CHUNK)
        pltpu.sync_copy(idx_hbm.at[wid], idx_v)

        for jj in range(LA):
            pltpu.async_copy(table_hbm.at[idx_v.at[jj]], rows_v.at[jj], gsem[jj])

        def outer(g, _):
            j0 = g * NBUF
            for bb in range(NBUF):
                j = j0 + bb
                jl = j + LA
                bl = (bb + LA) % NBUF

                @pl.when(jl < n_chunks)
                def _():
                    @pl.when(jl >= NBUF)
                    def _():
                        pltpu.make_async_copy(
                            rows_v.at[bl],
                            out_hbm.at[pl.ds(base, CHUNK), pl.ds(0, d)],
                            ssem[bl],
                        ).wait()

                    pltpu.async_copy(
                        table_hbm.at[idx_v.at[jl]], rows_v.at[bl], gsem[bl]
                    )

                pltpu.make_async_copy(
                    table_hbm.at[idx_v.at[j]], rows_v.at[bb], gsem[bb]
                ).wait()
                pltpu.async_copy(
                    rows_v.at[bb],
                    out_hbm.at[pl.ds(base + j * CHUNK, CHUNK), pl.ds(0, d)],
                    ssem[bb],
                )
            return 0

        lax.fori_loop(0, n_chunks // NBUF, outer, 0)

        for bb in range(NBUF):
            pltpu.make_async_copy(
                rows_v.at[bb], out_hbm.at[pl.ds(base, CHUNK), pl.ds(0, d)], ssem[bb]
            ).wait()

    return k(table, idx3)


def kernel(token_ids, weight):
    bt, s = token_ids.shape
    v, d = weight.shape
    flat = token_ids.reshape(-1).astype(jnp.int32)
    b = flat.shape[0]
    nw = 32
    per_w = b // nw
    n_chunks = per_w // CHUNK
    idx3 = (flat * 2).reshape(nw, n_chunks, CHUNK)
    table = jnp.pad(weight, ((0, 0), (0, 128 - d))).reshape(2 * v, d)
    out = _sc_gather(table, idx3, n_chunks=n_chunks, d=d)
    return out[:, :d].reshape(bt, s, d)
